# baseline (speedup 1.0000x reference)
import functools

import jax
import jax.numpy as jnp
from jax import lax
from jax.experimental import pallas as pl
from jax.experimental.pallas import tpu as pltpu
from jax.experimental.pallas import tpu_sc as plsc

_CHUNK = 128
_NW = 32


@functools.partial(jax.jit, static_argnames=("J", "Bt", "D"))
def _embed(xt3, table, *, J, Bt, D):
    bw = Bt // _NW
    hb = bw // _CHUNK
    nchunk = J * hb

    mesh = plsc.VectorSubcoreMesh(core_axis_name="c", subcore_axis_name="s")

    @functools.partial(
        pl.kernel,
        mesh=mesh,
        out_type=jax.ShapeDtypeStruct((J, D, Bt), jnp.float32),
        compiler_params=pltpu.CompilerParams(use_tc_tiling_on_sc=False,
                                             needs_layout_passes=False),
        scratch_types=(
            [pltpu.VMEM((J, hb, _CHUNK), jnp.int32),
             pltpu.VMEM((2, _CHUNK, D), jnp.float32),
             pltpu.VMEM((2, D, bw), jnp.float32)]
            + [pltpu.SemaphoreType.DMA] * 4
        ),
    )
    def emb(table_hbm, x_hbm, out_hbm, idx_v, gbuf, tbuf, *sems):
        gsem = sems[:2]
        osem = sems[2:]
        wid = lax.axis_index("s") * 2 + lax.axis_index("c")
        b0 = wid * bw
        pltpu.sync_copy(x_hbm.at[:, pl.ds(wid * hb, hb), :], idx_v)

        def fire_gather(i, p):
            j = i // hb
            h = i - j * hb
            pltpu.async_copy(table_hbm.at[idx_v.at[j, h]], gbuf.at[p],
                             gsem[p])

        def wait_gather(i, p):
            j = i // hb
            h = i - j * hb
            pltpu.make_async_copy(table_hbm.at[idx_v.at[j, h]], gbuf.at[p],
                                  gsem[p]).wait()

        def fire_out(j, p2):
            pltpu.async_copy(tbuf.at[p2],
                             out_hbm.at[j, :, pl.ds(b0, bw)], osem[p2])

        def wait_out(j, p2):
            pltpu.make_async_copy(tbuf.at[p2],
                                  out_hbm.at[j, :, pl.ds(b0, bw)],
                                  osem[p2]).wait()

        row_iota = lax.iota(jnp.int32, 16)

        def transpose(p, p2, q):
            src = gbuf.at[p]
            dst = tbuf.at[p2]

            @plsc.parallel_loop(0, D, unroll=4)
            def _(c):
                cc = jnp.zeros((16,), jnp.int32) + c
                for kk in range(_CHUNK // 16):
                    v = plsc.load_gather(src, [row_iota + (kk * 16), cc])
                    dst[c, pl.ds(q * _CHUNK + kk * 16, 16)] = v

        def jbody(j, p2, first):
            if not first:
                wait_out(j - 2, p2)
            for q in range(hb):
                i = j * hb + q
                p = q % 2
                wait_gather(i, p)
                transpose(p, p2, q)
                if not isinstance(j, int) or j * hb + q + 2 < nchunk:
                    fire_gather(i + 2, p)
            fire_out(j, p2)

        fire_gather(0, 0)
        fire_gather(1, 1)
        jbody(0, 0, True)
        jbody(1, 1, True)

        def group(g, carry):
            j = g * 2
            jbody(j, 0, False)
            jbody(j + 1, 1, False)
            return carry

        lax.fori_loop(1, (J - 2) // 2, group, 0)

        jbody(J - 2, 0, False)
        jbody(J - 1, 1, False)
        wait_out(J - 2, 0)
        wait_out(J - 1, 1)

    return emb(table, xt3)


def kernel(x, table):
    B, J = x.shape
    D = table.shape[1]
    xt = jnp.transpose(x).astype(jnp.int32)
    xt3 = xt.reshape(J, B // _CHUNK, _CHUNK)
    out_phys = _embed(xt3, table, J=J, Bt=B, D=D)
    return jnp.transpose(out_phys, (2, 0, 1))


# speedup vs baseline: 1.0013x; 1.0013x over previous
import functools

import jax
import jax.numpy as jnp
from jax import lax
from jax.experimental import pallas as pl
from jax.experimental.pallas import tpu as pltpu
from jax.experimental.pallas import tpu_sc as plsc

_CHUNK = 128
_NW = 32


@functools.partial(jax.jit, static_argnames=("J", "Bt", "D"))
def _embed(xt3, table, *, J, Bt, D):
    bw = Bt // _NW
    hb = bw // _CHUNK
    nchunk = J * hb

    mesh = plsc.VectorSubcoreMesh(core_axis_name="c", subcore_axis_name="s")

    @functools.partial(
        pl.kernel,
        mesh=mesh,
        out_type=jax.ShapeDtypeStruct((J, D, Bt), jnp.float32),
        compiler_params=pltpu.CompilerParams(use_tc_tiling_on_sc=False,
                                             needs_layout_passes=False),
        scratch_types=(
            [pltpu.VMEM((J, hb, _CHUNK), jnp.int32),
             pltpu.VMEM((2, _CHUNK, D), jnp.float32),
             pltpu.VMEM((2, D, bw), jnp.float32)]
            + [pltpu.SemaphoreType.DMA] * 4
        ),
    )
    def emb(table_hbm, x_hbm, out_hbm, idx_v, gbuf, tbuf, *sems):
        gsem = sems[:2]
        osem = sems[2:]
        wid = lax.axis_index("s") * 2 + lax.axis_index("c")
        b0 = wid * bw
        pltpu.sync_copy(x_hbm.at[:, pl.ds(wid * hb, hb), :], idx_v)

        def fire_gather(i, p):
            j = i // hb
            h = i - j * hb
            pltpu.async_copy(table_hbm.at[idx_v.at[j, h]], gbuf.at[p],
                             gsem[p])

        def wait_gather(i, p):
            j = i // hb
            h = i - j * hb
            pltpu.make_async_copy(table_hbm.at[idx_v.at[j, h]], gbuf.at[p],
                                  gsem[p]).wait()

        def fire_out(j, p2):
            pltpu.async_copy(tbuf.at[p2],
                             out_hbm.at[j, :, pl.ds(b0, bw)], osem[p2])

        def wait_out(j, p2):
            pltpu.make_async_copy(tbuf.at[p2],
                                  out_hbm.at[j, :, pl.ds(b0, bw)],
                                  osem[p2]).wait()

        row_iota = lax.iota(jnp.int32, 16)
        rows_kk = [row_iota + (kk * 16) for kk in range(_CHUNK // 16)]

        def transpose(p, p2, q):
            src = gbuf.at[p]
            dst = tbuf.at[p2]

            @plsc.parallel_loop(0, D, unroll=4)
            def _(c):
                cc = jnp.zeros((16,), jnp.int32) + c
                for kk in range(_CHUNK // 16):
                    v = plsc.load_gather(src, [rows_kk[kk], cc])
                    dst[c, pl.ds(q * _CHUNK + kk * 16, 16)] = v

        def jbody(j, p2, first):
            if not first:
                wait_out(j - 2, p2)
            for q in range(hb):
                i = j * hb + q
                p = q % 2
                wait_gather(i, p)
                transpose(p, p2, q)
                if not isinstance(j, int) or j * hb + q + 2 < nchunk:
                    fire_gather(i + 2, p)
            fire_out(j, p2)

        fire_gather(0, 0)
        fire_gather(1, 1)
        jbody(0, 0, True)
        jbody(1, 1, True)

        def group(g, carry):
            j = g * 2
            jbody(j, 0, False)
            jbody(j + 1, 1, False)
            return carry

        lax.fori_loop(1, (J - 2) // 2, group, 0)

        jbody(J - 2, 0, False)
        jbody(J - 1, 1, False)
        wait_out(J - 2, 0)
        wait_out(J - 1, 1)

    return emb(table, xt3)


def kernel(x, table):
    B, J = x.shape
    D = table.shape[1]
    xt = jnp.transpose(x).astype(jnp.int32)
    xt3 = xt.reshape(J, B // _CHUNK, _CHUNK)
    out_phys = _embed(xt3, table, J=J, Bt=B, D=D)
    return jnp.transpose(out_phys, (2, 0, 1))


# speedup vs baseline: 1.0014x; 1.0001x over previous
import functools

import jax
import jax.numpy as jnp
from jax import lax
from jax.experimental import pallas as pl
from jax.experimental.pallas import tpu as pltpu
from jax.experimental.pallas import tpu_sc as plsc

_CHUNK = 128
_NW = 32


@functools.partial(jax.jit, static_argnames=("J", "Bt", "D"))
def _embed(xt3, table, *, J, Bt, D):
    bw = Bt // _NW
    hb = bw // _CHUNK
    nchunk = J * hb

    mesh = plsc.VectorSubcoreMesh(core_axis_name="c", subcore_axis_name="s")

    @functools.partial(
        pl.kernel,
        mesh=mesh,
        out_type=jax.ShapeDtypeStruct((J, D, Bt), jnp.float32),
        compiler_params=pltpu.CompilerParams(use_tc_tiling_on_sc=False,
                                             needs_layout_passes=False),
        scratch_types=(
            [pltpu.VMEM((J, hb, _CHUNK), jnp.int32),
             pltpu.VMEM((4, _CHUNK, D), jnp.float32),
             pltpu.VMEM((2, D, bw), jnp.float32)]
            + [pltpu.SemaphoreType.DMA] * 6
        ),
    )
    def emb(table_hbm, x_hbm, out_hbm, idx_v, gbuf, tbuf, *sems):
        gsem = sems[:4]
        osem = sems[4:]
        wid = lax.axis_index("s") * 2 + lax.axis_index("c")
        b0 = wid * bw
        pltpu.sync_copy(x_hbm.at[:, pl.ds(wid * hb, hb), :], idx_v)

        def fire_gather(i, p):
            j = i // hb
            h = i - j * hb
            pltpu.async_copy(table_hbm.at[idx_v.at[j, h]], gbuf.at[p],
                             gsem[p])

        def wait_gather(i, p):
            j = i // hb
            h = i - j * hb
            pltpu.make_async_copy(table_hbm.at[idx_v.at[j, h]], gbuf.at[p],
                                  gsem[p]).wait()

        def fire_out(j, p2):
            pltpu.async_copy(tbuf.at[p2],
                             out_hbm.at[j, :, pl.ds(b0, bw)], osem[p2])

        def wait_out(j, p2):
            pltpu.make_async_copy(tbuf.at[p2],
                                  out_hbm.at[j, :, pl.ds(b0, bw)],
                                  osem[p2]).wait()

        row_iota = lax.iota(jnp.int32, 16)
        rows_kk = [row_iota + (kk * 16) for kk in range(_CHUNK // 16)]

        def transpose(p, p2, q):
            src = gbuf.at[p]
            dst = tbuf.at[p2]

            @plsc.parallel_loop(0, D, unroll=4)
            def _(c):
                cc = jnp.zeros((16,), jnp.int32) + c
                for kk in range(_CHUNK // 16):
                    v = plsc.load_gather(src, [rows_kk[kk], cc])
                    dst[c, pl.ds(q * _CHUNK + kk * 16, 16)] = v

        def jbody(j, p2, first):
            if not first:
                wait_out(j - 2, p2)
            for q in range(hb):
                i = j * hb + q
                p = q
                wait_gather(i, p)
                transpose(p, p2, q)
                if not isinstance(j, int) or j * hb + q + 4 < nchunk:
                    fire_gather(i + 4, p)
            fire_out(j, p2)

        for p in range(4):
            fire_gather(p, p)
        jbody(0, 0, True)
        jbody(1, 1, True)

        def group(g, carry):
            j = g * 2
            jbody(j, 0, False)
            jbody(j + 1, 1, False)
            return carry

        lax.fori_loop(1, (J - 2) // 2, group, 0)

        jbody(J - 2, 0, False)
        jbody(J - 1, 1, False)
        wait_out(J - 2, 0)
        wait_out(J - 1, 1)

    return emb(table, xt3)


def kernel(x, table):
    B, J = x.shape
    D = table.shape[1]
    xt = jnp.transpose(x).astype(jnp.int32)
    xt3 = xt.reshape(J, B // _CHUNK, _CHUNK)
    out_phys = _embed(xt3, table, J=J, Bt=B, D=D)
    return jnp.transpose(out_phys, (2, 0, 1))


# speedup vs baseline: 1.4869x; 1.4848x over previous
import functools

import jax
import jax.numpy as jnp
from jax import lax
from jax.experimental import pallas as pl
from jax.experimental.pallas import tpu as pltpu
from jax.experimental.pallas import tpu_sc as plsc

_CHUNK = 128
_NW = 32


@functools.partial(jax.jit, static_argnames=("J", "Bt", "D"))
def _embed(xt3, table, *, J, Bt, D):
    bw = Bt // _NW
    hb = bw // _CHUNK
    nchunk = J * hb

    mesh = plsc.VectorSubcoreMesh(core_axis_name="c", subcore_axis_name="s")

    @functools.partial(
        pl.kernel,
        mesh=mesh,
        out_type=jax.ShapeDtypeStruct((J, D, Bt), jnp.float32),
        compiler_params=pltpu.CompilerParams(use_tc_tiling_on_sc=False,
                                             needs_layout_passes=False),
        scratch_types=(
            [pltpu.VMEM((J, hb, _CHUNK), jnp.int32),
             pltpu.VMEM((4, _CHUNK, D), jnp.float32),
             pltpu.VMEM((2, D, bw), jnp.float32)]
            + [pltpu.SemaphoreType.DMA] * 6
        ),
    )
    def emb(table_hbm, x_hbm, out_hbm, idx_v, gbuf, tbuf, *sems):
        gsem = sems[:4]
        osem = sems[4:]
        wid = lax.axis_index("s") * 2 + lax.axis_index("c")
        b0 = wid * bw
        pltpu.sync_copy(x_hbm.at[:, pl.ds(wid * hb, hb), :], idx_v)

        def fire_gather(i, p):
            j = i // hb
            h = i - j * hb
            pltpu.async_copy(table_hbm.at[idx_v.at[j, h]], gbuf.at[p],
                             gsem[p])

        def wait_gather(i, p):
            j = i // hb
            h = i - j * hb
            pltpu.make_async_copy(table_hbm.at[idx_v.at[j, h]], gbuf.at[p],
                                  gsem[p]).wait()

        def fire_out(j, p2):
            pltpu.async_copy(tbuf.at[p2],
                             out_hbm.at[j, :, pl.ds(b0, bw)], osem[p2])

        def wait_out(j, p2):
            pltpu.make_async_copy(tbuf.at[p2],
                                  out_hbm.at[j, :, pl.ds(b0, bw)],
                                  osem[p2]).wait()

        row_iota = lax.iota(jnp.int32, 16)
        rows_kk = [row_iota + (kk * 16) for kk in range(_CHUNK // 16)]

        def transpose(p, p2, q):
            src = gbuf.at[p]
            dst = tbuf.at[p2]
            dstcols = [row_iota + (q * _CHUNK + kk * 16)
                       for kk in range(_CHUNK // 16)]

            @plsc.parallel_loop(0, 16, unroll=1)
            def _(s):
                perm = (row_iota + s) & 15
                for cg in range(D // 16):
                    colv = perm + (cg * 16)
                    for kk in range(_CHUNK // 16):
                        v = plsc.load_gather(src, [rows_kk[kk], colv])
                        plsc.store_scatter(dst, [colv, dstcols[kk]], v)

        def jbody(j, p2, first):
            if not first:
                wait_out(j - 2, p2)
            for q in range(hb):
                i = j * hb + q
                p = q
                wait_gather(i, p)
                transpose(p, p2, q)
                if isinstance(j, int):
                    if j * hb + q + 4 < nchunk:
                        fire_gather(i + 4, p)
                else:
                    @pl.when(i + 4 < nchunk)
                    def _():
                        fire_gather(i + 4, p)
            fire_out(j, p2)

        for p in range(4):
            fire_gather(p, p)
        jbody(0, 0, True)
        jbody(1, 1, True)

        def group(g, carry):
            j = g * 2
            jbody(j, 0, False)
            jbody(j + 1, 1, False)
            return carry

        lax.fori_loop(1, J // 2, group, 0)

        wait_out(J - 2, 0)
        wait_out(J - 1, 1)

    return emb(table, xt3)


def kernel(x, table):
    B, J = x.shape
    D = table.shape[1]
    xt = jnp.transpose(x).astype(jnp.int32)
    xt3 = xt.reshape(J, B // _CHUNK, _CHUNK)
    out_phys = _embed(xt3, table, J=J, Bt=B, D=D)
    return jnp.transpose(out_phys, (2, 0, 1))
